# Initial kernel scaffold; baseline (speedup 1.0000x reference)
import jax
import jax.numpy as jnp
from jax.experimental import pallas as pl


def kernel(depth, pred, idx_A, idx_B):
    raise NotImplementedError("write your pallas kernel here")


# speedup vs baseline: 1.2643x; 1.2643x over previous
import functools

import jax
import jax.numpy as jnp
from jax import lax
from jax.experimental import pallas as pl
from jax.experimental.pallas import tpu as pltpu
from jax.experimental.pallas import tpu_sc as plsc

THETA_F = 1.15
FILTER_F = 1e-08
NEG_BIG = -1e30

NC = 2
NS = 16
NW = NC * NS
LANES = 16
CHUNK = 128


def _sc_gather_logits(n_rows_per_w):
    mesh = plsc.VectorSubcoreMesh(core_axis_name="c", subcore_axis_name="s")
    total_rows = NW * n_rows_per_w

    @functools.partial(
        pl.kernel,
        mesh=mesh,
        out_type=jax.ShapeDtypeStruct((NW, n_rows_per_w, CHUNK), jnp.float32),
        scratch_types=[
            pltpu.VMEM((n_rows_per_w, CHUNK), jnp.int32),
            pltpu.VMEM((n_rows_per_w, CHUNK), jnp.int32),
            pltpu.VMEM((n_rows_per_w, CHUNK), jnp.float32),
            pltpu.VMEM((n_rows_per_w, CHUNK), jnp.float32),
            pltpu.VMEM((n_rows_per_w, CHUNK), jnp.float32),
            pltpu.VMEM((n_rows_per_w, CHUNK), jnp.float32),
            pltpu.VMEM((n_rows_per_w, CHUNK), jnp.float32),
            pltpu.SemaphoreType.DMA,
        ],
    )
    def sc_kernel(d_hbm, p_hbm, ia_hbm, ib_hbm, x_hbm,
                  ia_v, ib_v, za_v, zb_v, pa_v, pb_v, x_v, sem):
        wid = lax.axis_index("s") * NC + lax.axis_index("c")
        pltpu.sync_copy(ia_hbm.at[wid], ia_v)
        pltpu.sync_copy(ib_hbm.at[wid], ib_v)
        handles = []
        for j in range(n_rows_per_w):
            handles.append(pltpu.async_copy(d_hbm.at[ia_v.at[j]], za_v.at[j], sem))
            handles.append(pltpu.async_copy(d_hbm.at[ib_v.at[j]], zb_v.at[j], sem))
            handles.append(pltpu.async_copy(p_hbm.at[ia_v.at[j]], pa_v.at[j], sem))
            handles.append(pltpu.async_copy(p_hbm.at[ib_v.at[j]], pb_v.at[j], sem))
        for h in handles:
            h.wait()
        one = jnp.float32(1.0)
        neg_one = jnp.float32(-1.0)
        zero = jnp.float32(0.0)
        for j in range(n_rows_per_w):
            for k in range(CHUNK // LANES):
                s = pl.ds(k * LANES, LANES)
                za = za_v[j, s]
                zb = zb_v[j, s]
                pa = pa_v[j, s]
                pb = pb_v[j, s]
                keep = (za > FILTER_F) | (zb > FILTER_F)
                t = jnp.where(za / zb > THETA_F, neg_one,
                              jnp.where(zb / za > THETA_F, one, zero))
                valid = keep & (t != zero)
                x = jnp.where(valid, -t * (pa - pb), jnp.float32(NEG_BIG))
                x_v[j, s] = x
        pltpu.sync_copy(x_v, x_hbm.at[wid])

    return sc_kernel


def _softplus_sum(x_ref, o_ref):
    x = x_ref[...]
    o_ref[...] = jnp.sum(jnp.log1p(jnp.exp(x))).reshape(1, 1)


def kernel(depth, pred, idx_A, idx_B):
    d = depth.reshape(-1)
    p = pred.reshape(-1)
    n = idx_A.shape[0]
    n_rows_per_w = n // (NW * CHUNK)
    total_rows = NW * n_rows_per_w
    ia = idx_A.reshape(NW, n_rows_per_w, CHUNK)
    ib = idx_B.reshape(NW, n_rows_per_w, CHUNK)
    x = _sc_gather_logits(n_rows_per_w)(d, p, ia, ib)
    loss = pl.pallas_call(
        _softplus_sum,
        out_shape=jax.ShapeDtypeStruct((1, 1), jnp.float32),
    )(x.reshape(total_rows, CHUNK))
    return loss[0, 0]


# speedup vs baseline: 1.3054x; 1.0325x over previous
import functools

import jax
import jax.numpy as jnp
from jax import lax
from jax.experimental import pallas as pl
from jax.experimental.pallas import tpu as pltpu
from jax.experimental.pallas import tpu_sc as plsc

THETA_F = 1.15
FILTER_F = 1e-08
NEG_BIG = -1e30

NC = 2
NS = 16
NW = NC * NS
LANES = 16
CHUNK = 128


def _sc_gather_logits(n_per_w):
    mesh = plsc.VectorSubcoreMesh(core_axis_name="c", subcore_axis_name="s")
    n_chunks = n_per_w // CHUNK

    @functools.partial(
        pl.kernel,
        mesh=mesh,
        out_type=jax.ShapeDtypeStruct((NW * n_per_w,), jnp.float32),
        scratch_types=[
            pltpu.VMEM((n_per_w,), jnp.int32),
            pltpu.VMEM((n_per_w,), jnp.int32),
            pltpu.VMEM((n_per_w,), jnp.float32),
            pltpu.VMEM((n_per_w,), jnp.float32),
            pltpu.VMEM((n_per_w,), jnp.float32),
            pltpu.VMEM((n_per_w,), jnp.float32),
            pltpu.VMEM((n_per_w,), jnp.float32),
            pltpu.SemaphoreType.DMA,
        ],
    )
    def sc_kernel(d_hbm, p_hbm, ia_hbm, ib_hbm, x_hbm,
                  ia_v, ib_v, za_v, zb_v, pa_v, pb_v, x_v, sem):
        wid = lax.axis_index("s") * NC + lax.axis_index("c")
        base = wid * n_per_w
        pltpu.sync_copy(ia_hbm.at[pl.ds(base, n_per_w)], ia_v)
        pltpu.sync_copy(ib_hbm.at[pl.ds(base, n_per_w)], ib_v)
        handles = []
        for j in range(n_chunks):
            s = pl.ds(j * CHUNK, CHUNK)
            handles.append(pltpu.async_copy(d_hbm.at[ia_v.at[s]], za_v.at[s], sem))
            handles.append(pltpu.async_copy(d_hbm.at[ib_v.at[s]], zb_v.at[s], sem))
            handles.append(pltpu.async_copy(p_hbm.at[ia_v.at[s]], pa_v.at[s], sem))
            handles.append(pltpu.async_copy(p_hbm.at[ib_v.at[s]], pb_v.at[s], sem))
        for h in handles:
            h.wait()
        one = jnp.float32(1.0)
        neg_one = jnp.float32(-1.0)
        zero = jnp.float32(0.0)
        for k in range(n_per_w // LANES):
            s = pl.ds(k * LANES, LANES)
            za = za_v[s]
            zb = zb_v[s]
            pa = pa_v[s]
            pb = pb_v[s]
            keep = (za > FILTER_F) | (zb > FILTER_F)
            t = jnp.where(za / zb > THETA_F, neg_one,
                          jnp.where(zb / za > THETA_F, one, zero))
            valid = keep & (t != zero)
            x = jnp.where(valid, -t * (pa - pb), jnp.float32(NEG_BIG))
            x_v[s] = x
        pltpu.sync_copy(x_v, x_hbm.at[pl.ds(base, n_per_w)])

    return sc_kernel


def _softplus_sum(x_ref, o_ref):
    x = x_ref[...]
    o_ref[...] = jnp.sum(jnp.log1p(jnp.exp(x))).reshape(1, 1)


def kernel(depth, pred, idx_A, idx_B):
    d = depth.reshape(-1)
    p = pred.reshape(-1)
    n = idx_A.shape[0]
    n_per_w = n // NW
    x = _sc_gather_logits(n_per_w)(d, p, idx_A, idx_B)
    loss = pl.pallas_call(
        _softplus_sum,
        out_shape=jax.ShapeDtypeStruct((1, 1), jnp.float32),
    )(x.reshape(n // CHUNK, CHUNK))
    return loss[0, 0]
